# baseline (speedup 1.0000x reference)
import functools

import jax
import jax.numpy as jnp
from jax import lax
from jax.experimental import pallas as pl
from jax.experimental.pallas import tpu as pltpu
from jax.experimental.pallas import tpu_sc as plsc

N_NODES = 10000
N_PAD = 10240
D = 128
H = 8
DH = 16

NC = 2
NS = 16
NW = NC * NS

ZROWS = N_PAD // 16

CHUNK = 40
SUP = 10
IO_ROWS = 40


def _sc_edge_kernel(q_hbm, k_hbm, v_hbm, src_hbm, dst_hbm,
                    out_wv, out_z,
                    srcb, dstb, kqv0, kqv1, vbuf, contrib, contrib_z,
                    svbuf, dst_sc, zrow_v, acc_wv, acc_z,
                    sem_g0, sem_g1, sem_v, sem_s, sem_z):
    c = lax.axis_index("c")
    s = lax.axis_index("s")
    w = s * NC + c

    e_total = src_hbm.shape[0]
    e_per_tile = e_total // NW
    n_super = e_per_tile // (SUP * CHUNK)
    rows_per_tile = N_PAD // NS
    zrows_per_tile = ZROWS // NS

    lane = lax.iota(jnp.int32, 16)
    zero16 = jnp.zeros((16,), jnp.float32)
    kq = (kqv0, kqv1)
    sem_g = (sem_g0, sem_g1)
    iobuf = kqv0

    @pl.loop(0, IO_ROWS * 8)
    def _z(i):
        iobuf[i // 8, pl.ds((i % 8) * 16, 16)] = zero16

    @pl.loop(0, CHUNK * 8)
    def _zc(i):
        contrib_z[i // 8, pl.ds((i % 8) * 16, 16)] = zero16

    io0 = iobuf.at[pl.ds(0, IO_ROWS)]

    @pl.loop(0, rows_per_tile // IO_ROWS)
    def _z2(j):
        pltpu.sync_copy(io0, acc_wv.at[pl.ds(s * rows_per_tile + j * IO_ROWS, IO_ROWS)])

    pltpu.sync_copy(iobuf.at[pl.ds(0, zrows_per_tile)],
                    acc_z.at[pl.ds(s * zrows_per_tile, zrows_per_tile)])

    plsc.subcore_barrier()

    row_off = lane // H
    head = lane % H
    colb = head * DH
    cols = tuple(colb + d for d in range(DH))
    ebase0 = w * e_per_tile

    def issue_gathers(sub, sbase):
        b = sub % 2
        buf = kq[b]
        sem = sem_g[b]
        so = sub * CHUNK
        ck = pltpu.async_copy(k_hbm.at[srcb.at[pl.ds(so, CHUNK)]],
                              buf.at[pl.ds(0, CHUNK)], sem)
        cq = pltpu.async_copy(q_hbm.at[dstb.at[pl.ds(so, CHUNK)]],
                              buf.at[pl.ds(CHUNK, CHUNK)], sem)
        return (ck, cq)

    def compute_chunk(sub):
        b = sub % 2
        buf = kq[b]
        kbuf = buf.at[pl.ds(0, CHUNK)]
        qbuf = buf.at[pl.ds(CHUNK, CHUNK)]
        so = sub * CHUNK
        cv = pltpu.async_copy(v_hbm.at[srcb.at[pl.ds(so, CHUNK)]],
                              vbuf, sem_v)
        for t in range(CHUNK // 16):
            dv = dstb[pl.ds(so + t * 16, 16)]
            dst_sc[pl.ds(t * 16, 16)] = dv
            zrow_v[pl.ds(t * 16, 16)] = dv >> 4
        rem = CHUNK % 16
        if rem:
            ridx = jnp.minimum(lane + (CHUNK // 16) * 16, CHUNK - 1)
            dvr = plsc.load_gather(dstb, [so + ridx])
            plsc.store_scatter(dst_sc, [ridx], dvr, mask=lane < rem)
            plsc.store_scatter(zrow_v, [ridx], dvr >> 4, mask=lane < rem)

        @pl.loop(0, CHUNK // 4)
        def _grp(i):
            rows0 = row_off + 4 * i
            rows1 = row_off + 4 * i + 2
            sa0 = jnp.zeros((16,), jnp.float32)
            sb0 = jnp.zeros((16,), jnp.float32)
            sa1 = jnp.zeros((16,), jnp.float32)
            sb1 = jnp.zeros((16,), jnp.float32)
            for d in range(0, DH, 2):
                sa0 = sa0 + (plsc.load_gather(kbuf, [rows0, cols[d]])
                             * plsc.load_gather(qbuf, [rows0, cols[d]]))
                sa1 = sa1 + (plsc.load_gather(kbuf, [rows1, cols[d]])
                             * plsc.load_gather(qbuf, [rows1, cols[d]]))
                sb0 = sb0 + (plsc.load_gather(kbuf, [rows0, cols[d + 1]])
                             * plsc.load_gather(qbuf, [rows0, cols[d + 1]]))
                sb1 = sb1 + (plsc.load_gather(kbuf, [rows1, cols[d + 1]])
                             * plsc.load_gather(qbuf, [rows1, cols[d + 1]]))
            sval0 = jnp.exp(jnp.clip((sa0 + sb0) * 0.25, -5.0, 5.0))
            sval1 = jnp.exp(jnp.clip((sa1 + sb1) * 0.25, -5.0, 5.0))
            svbuf[pl.ds(i * 32, 16)] = sval0
            svbuf[pl.ds(i * 32 + 16, 16)] = sval1

        cv.wait()

        @pl.loop(0, CHUNK // 4)
        def _vph(i):
            rows0 = row_off + 4 * i
            rows1 = row_off + 4 * i + 2
            sval0 = svbuf[pl.ds(i * 32, 16)]
            sval1 = svbuf[pl.ds(i * 32 + 16, 16)]
            for d in range(DH):
                v0 = plsc.load_gather(vbuf, [rows0, cols[d]])
                v1 = plsc.load_gather(vbuf, [rows1, cols[d]])
                plsc.store_scatter(contrib, [rows0, cols[d]], v0 * sval0)
                plsc.store_scatter(contrib, [rows1, cols[d]], v1 * sval1)

        @pl.loop(0, CHUNK)
        def _zed(e):
            dsp = plsc.load_gather(dst_sc, [jnp.full((16,), e, jnp.int32)])
            for j in range(8):
                contrib_z[e, pl.ds(j * 16, 16)] = zero16
            sv = svbuf[pl.ds(e * 8, 16)]
            plsc.store_scatter(
                contrib_z,
                [jnp.full((16,), e, jnp.int32), (dsp & 15) * 8 + lane],
                sv, mask=lane < 8)

        cw = pltpu.async_copy(contrib, acc_wv.at[dst_sc], sem_s, add=True)
        cz = pltpu.async_copy(contrib_z, acc_z.at[zrow_v], sem_z, add=True)
        cw.wait()
        cz.wait()

    @pl.loop(0, n_super)
    def _sup(sp):
        sbase = ebase0 + sp * (SUP * CHUNK)
        pltpu.sync_copy(src_hbm.at[pl.ds(sbase, SUP * CHUNK)],
                        srcb.at[pl.ds(0, SUP * CHUNK)])
        pltpu.sync_copy(dst_hbm.at[pl.ds(sbase, SUP * CHUNK)],
                        dstb.at[pl.ds(0, SUP * CHUNK)])
        pend = issue_gathers(0, sbase)
        for sub in range(SUP):
            nxt = issue_gathers(sub + 1, sbase) if sub + 1 < SUP else None
            for d_ in pend:
                d_.wait()
            pend = nxt
            compute_chunk(sub)

    plsc.subcore_barrier()

    @pl.loop(0, rows_per_tile // IO_ROWS)
    def _out(j):
        r0 = s * rows_per_tile + j * IO_ROWS
        pltpu.sync_copy(acc_wv.at[pl.ds(r0, IO_ROWS)], io0)
        pltpu.sync_copy(io0, out_wv.at[c, pl.ds(r0, IO_ROWS)])

    zr0 = s * zrows_per_tile
    pltpu.sync_copy(acc_z.at[pl.ds(zr0, zrows_per_tile)],
                    iobuf.at[pl.ds(0, zrows_per_tile)])
    pltpu.sync_copy(iobuf.at[pl.ds(0, zrows_per_tile)],
                    out_z.at[c, pl.ds(zr0, zrows_per_tile)])


def _edge_aggregate(q, k, v, src, dst):
    fn = pl.kernel(
        _sc_edge_kernel,
        out_type=[
            jax.ShapeDtypeStruct((NC, N_PAD, D), jnp.float32),
            jax.ShapeDtypeStruct((NC, ZROWS, D), jnp.float32),
        ],
        mesh=plsc.VectorSubcoreMesh(core_axis_name="c", subcore_axis_name="s",
                                    num_cores=NC, num_subcores=NS),
        compiler_params=pltpu.CompilerParams(needs_layout_passes=False),
        scratch_types=[
            pltpu.VMEM((SUP * CHUNK + 16,), jnp.int32),
            pltpu.VMEM((SUP * CHUNK + 16,), jnp.int32),
            pltpu.VMEM((2 * CHUNK, D), jnp.float32),
            pltpu.VMEM((2 * CHUNK, D), jnp.float32),
            pltpu.VMEM((CHUNK, D), jnp.float32),
            pltpu.VMEM((CHUNK, D), jnp.float32),
            pltpu.VMEM((CHUNK, D), jnp.float32),
            pltpu.VMEM((CHUNK * 8 + 8,), jnp.float32),
            pltpu.VMEM((CHUNK,), jnp.int32),
            pltpu.VMEM((CHUNK,), jnp.int32),
            pltpu.VMEM_SHARED((N_PAD, D), jnp.float32),
            pltpu.VMEM_SHARED((ZROWS, D), jnp.float32),
            pltpu.SemaphoreType.DMA,
            pltpu.SemaphoreType.DMA,
            pltpu.SemaphoreType.DMA,
            pltpu.SemaphoreType.DMA,
            pltpu.SemaphoreType.DMA,
        ],
    )
    return fn(q, k, v, src, dst)


_BLK = 1000


def _qkv_body(h_ref, wqt_ref, wkt_ref, wvt_ref, q_ref, k_ref, v_ref):
    hb = h_ref[...]
    q_ref[...] = jnp.dot(hb, wqt_ref[...], preferred_element_type=jnp.float32)
    k_ref[...] = jnp.dot(hb, wkt_ref[...], preferred_element_type=jnp.float32)
    v_ref[...] = jnp.dot(hb, wvt_ref[...], preferred_element_type=jnp.float32)


def _qkv(h, wqt, wkt, wvt):
    n = h.shape[0]
    grid = n // _BLK
    return pl.pallas_call(
        _qkv_body,
        grid=(grid,),
        in_specs=[
            pl.BlockSpec((_BLK, D), lambda i: (i, 0)),
            pl.BlockSpec((D, D), lambda i: (0, 0)),
            pl.BlockSpec((D, D), lambda i: (0, 0)),
            pl.BlockSpec((D, D), lambda i: (0, 0)),
        ],
        out_specs=[
            pl.BlockSpec((_BLK, D), lambda i: (i, 0)),
            pl.BlockSpec((_BLK, D), lambda i: (i, 0)),
            pl.BlockSpec((_BLK, D), lambda i: (i, 0)),
        ],
        out_shape=[jax.ShapeDtypeStruct((n, D), jnp.float32)] * 3,
    )(h, wqt, wkt, wvt)


def _attn_out_body(wv_ref, z_ref, h_ref, r_ref, wot_ref, bo_ref, x1_ref,
                   st_ref):
    wv = wv_ref[0] + wv_ref[1]
    z = z_ref[0] + z_ref[1]
    zr = jnp.dot(z, r_ref[...], preferred_element_type=jnp.float32)
    head_out = wv / zr
    x = (jnp.dot(head_out, wot_ref[...], preferred_element_type=jnp.float32)
         + bo_ref[...] + h_ref[...])
    x1_ref[...] = x

    @pl.when(pl.program_id(0) == 0)
    def _():
        st_ref[...] = jnp.zeros_like(st_ref)

    st_ref[0:1, :] += jnp.sum(x, axis=0, keepdims=True)
    st_ref[1:2, :] += jnp.sum(x * x, axis=0, keepdims=True)


def _attn_out(acc_wv, zf, h, rmat, wot, bo2):
    n = h.shape[0]
    grid = n // _BLK
    return pl.pallas_call(
        _attn_out_body,
        grid=(grid,),
        in_specs=[
            pl.BlockSpec((NC, _BLK, D), lambda i: (0, i, 0)),
            pl.BlockSpec((NC, _BLK, H), lambda i: (0, i, 0)),
            pl.BlockSpec((_BLK, D), lambda i: (i, 0)),
            pl.BlockSpec((H, D), lambda i: (0, 0)),
            pl.BlockSpec((D, D), lambda i: (0, 0)),
            pl.BlockSpec((1, D), lambda i: (0, 0)),
        ],
        out_specs=[
            pl.BlockSpec((_BLK, D), lambda i: (i, 0)),
            pl.BlockSpec((8, D), lambda i: (0, 0)),
        ],
        out_shape=[
            jax.ShapeDtypeStruct((n, D), jnp.float32),
            jax.ShapeDtypeStruct((8, D), jnp.float32),
        ],
    )(acc_wv, zf, h, rmat, wot, bo2)


def _ffn_body(x1_ref, st_ref, g1_ref, be1_ref, w1t_ref, b1_ref, w2t_ref,
              b2_ref, x2_ref, st2_ref):
    inv_n = 1.0 / N_NODES
    mu = st_ref[0:1, :] * inv_n
    var = st_ref[1:2, :] * inv_n - mu * mu
    inv = lax.rsqrt(var + 1e-5)
    xn = (x1_ref[...] - mu) * (inv * g1_ref[...]) + be1_ref[...]
    t = jnp.maximum(
        jnp.dot(xn, w1t_ref[...], preferred_element_type=jnp.float32)
        + b1_ref[...], 0.0)
    y = jnp.dot(t, w2t_ref[...], preferred_element_type=jnp.float32) + b2_ref[...]
    x2 = xn + y
    x2_ref[...] = x2

    @pl.when(pl.program_id(0) == 0)
    def _():
        st2_ref[...] = jnp.zeros_like(st2_ref)

    st2_ref[0:1, :] += jnp.sum(x2, axis=0, keepdims=True)
    st2_ref[1:2, :] += jnp.sum(x2 * x2, axis=0, keepdims=True)


def _ffn(x1, st, g1b, be1b, w1t, b1b, w2t, b2b):
    n = x1.shape[0]
    grid = n // _BLK
    return pl.pallas_call(
        _ffn_body,
        grid=(grid,),
        in_specs=[
            pl.BlockSpec((_BLK, D), lambda i: (i, 0)),
            pl.BlockSpec((8, D), lambda i: (0, 0)),
            pl.BlockSpec((1, D), lambda i: (0, 0)),
            pl.BlockSpec((1, D), lambda i: (0, 0)),
            pl.BlockSpec((D, 2 * D), lambda i: (0, 0)),
            pl.BlockSpec((1, 2 * D), lambda i: (0, 0)),
            pl.BlockSpec((2 * D, D), lambda i: (0, 0)),
            pl.BlockSpec((1, D), lambda i: (0, 0)),
        ],
        out_specs=[
            pl.BlockSpec((_BLK, D), lambda i: (i, 0)),
            pl.BlockSpec((8, D), lambda i: (0, 0)),
        ],
        out_shape=[
            jax.ShapeDtypeStruct((n, D), jnp.float32),
            jax.ShapeDtypeStruct((8, D), jnp.float32),
        ],
    )(x1, st, g1b, be1b, w1t, b1b, w2t, b2b)


def _bn2_body(x2_ref, st2_ref, g2_ref, be2_ref, out_ref):
    inv_n = 1.0 / N_NODES
    mu = st2_ref[0:1, :] * inv_n
    var = st2_ref[1:2, :] * inv_n - mu * mu
    inv = lax.rsqrt(var + 1e-5)
    out_ref[...] = (x2_ref[...] - mu) * (inv * g2_ref[...]) + be2_ref[...]


def _bn2(x2, st2, g2b, be2b):
    n = x2.shape[0]
    grid = n // _BLK
    return pl.pallas_call(
        _bn2_body,
        grid=(grid,),
        in_specs=[
            pl.BlockSpec((_BLK, D), lambda i: (i, 0)),
            pl.BlockSpec((8, D), lambda i: (0, 0)),
            pl.BlockSpec((1, D), lambda i: (0, 0)),
            pl.BlockSpec((1, D), lambda i: (0, 0)),
        ],
        out_specs=pl.BlockSpec((_BLK, D), lambda i: (i, 0)),
        out_shape=jax.ShapeDtypeStruct((n, D), jnp.float32),
    )(x2, st2, g2b, be2b)


def kernel(h, edge_index, Wq, Wk, Wv, Wo, bo, W1, b1, W2, b2, g1, be1, g2, be2):
    wqt = Wq.T
    wkt = Wk.T
    wvt = Wv.T
    wot = Wo.T
    w1t = W1.T
    w2t = W2.T
    bo2 = bo.reshape(1, D)
    b1b = b1.reshape(1, 2 * D)
    b2b = b2.reshape(1, D)
    g1b = g1.reshape(1, D)
    be1b = be1.reshape(1, D)
    g2b = g2.reshape(1, D)
    be2b = be2.reshape(1, D)
    rmat = (jnp.arange(D, dtype=jnp.int32)[None, :] // DH
            == jnp.arange(H, dtype=jnp.int32)[:, None]).astype(jnp.float32)

    q, k, v = _qkv(h, wqt, wkt, wvt)
    acc_wv, acc_z = _edge_aggregate(q, k, v, edge_index[0], edge_index[1])
    zf = acc_z.reshape(NC, N_PAD, H)
    x1, st = _attn_out(acc_wv, zf, h, rmat, wot, bo2)
    x2, st2 = _ffn(x1, st, g1b, be1b, w1t, b1b, w2t, b2b)
    return _bn2(x2, st2, g2b, be2b)


# speedup vs baseline: 1.9387x; 1.9387x over previous
import functools

import jax
import jax.numpy as jnp
from jax import lax
from jax.experimental import pallas as pl
from jax.experimental.pallas import tpu as pltpu
from jax.experimental.pallas import tpu_sc as plsc

N_NODES = 10000
N_PAD = 10240
D = 128
H = 8
DH = 16

NC = 2
NS = 16
NW = NC * NS

ZROWS = N_PAD // 16

CHUNK = 40
SUP = 10
IO_ROWS = 40


def _sc_edge_kernel(q_hbm, k_hbm, v_hbm, src_hbm, dst_hbm,
                    out_wv, out_z,
                    srcb, dstb, kqv0, kqv1, vbuf, contrib, contrib_z,
                    svbuf, dst_sc, zrow_v, acc_wv, acc_z,
                    sem_g0, sem_g1, sem_v, sem_s, sem_z):
    c = lax.axis_index("c")
    s = lax.axis_index("s")
    w = s * NC + c

    e_total = src_hbm.shape[0]
    e_per_tile = e_total // NW
    n_super = e_per_tile // (SUP * CHUNK)
    rows_per_tile = N_PAD // NS
    zrows_per_tile = ZROWS // NS

    lane = lax.iota(jnp.int32, 16)
    zero16 = jnp.zeros((16,), jnp.float32)
    kq = (kqv0, kqv1)
    sem_g = (sem_g0, sem_g1)
    iobuf = kqv0

    @pl.loop(0, IO_ROWS * 8)
    def _z(i):
        iobuf[i // 8, pl.ds((i % 8) * 16, 16)] = zero16

    @pl.loop(0, CHUNK * 8)
    def _zc(i):
        contrib_z[i // 8, pl.ds((i % 8) * 16, 16)] = zero16

    io0 = iobuf.at[pl.ds(0, IO_ROWS)]

    @pl.loop(0, rows_per_tile // IO_ROWS)
    def _z2(j):
        pltpu.sync_copy(io0, acc_wv.at[pl.ds(s * rows_per_tile + j * IO_ROWS, IO_ROWS)])

    pltpu.sync_copy(iobuf.at[pl.ds(0, zrows_per_tile)],
                    acc_z.at[pl.ds(s * zrows_per_tile, zrows_per_tile)])

    plsc.subcore_barrier()

    row_off = lane // H
    head = lane % H
    cols = tuple(head + d * H for d in range(DH))
    ebase0 = w * e_per_tile

    def issue_gathers(sub, sbase):
        b = sub % 2
        buf = kq[b]
        sem = sem_g[b]
        so = sub * CHUNK
        ck = pltpu.async_copy(k_hbm.at[srcb.at[pl.ds(so, CHUNK)]],
                              buf.at[pl.ds(0, CHUNK)], sem)
        cq = pltpu.async_copy(q_hbm.at[dstb.at[pl.ds(so, CHUNK)]],
                              buf.at[pl.ds(CHUNK, CHUNK)], sem)
        return (ck, cq)

    def compute_chunk(sub):
        b = sub % 2
        buf = kq[b]
        kbuf = buf.at[pl.ds(0, CHUNK)]
        qbuf = buf.at[pl.ds(CHUNK, CHUNK)]
        so = sub * CHUNK
        cv = pltpu.async_copy(v_hbm.at[srcb.at[pl.ds(so, CHUNK)]],
                              vbuf, sem_v)
        for t in range(CHUNK // 16):
            dv = dstb[pl.ds(so + t * 16, 16)]
            dst_sc[pl.ds(t * 16, 16)] = dv
            zrow_v[pl.ds(t * 16, 16)] = dv >> 4
        rem = CHUNK % 16
        if rem:
            ridx = jnp.minimum(lane + (CHUNK // 16) * 16, CHUNK - 1)
            dvr = plsc.load_gather(dstb, [so + ridx])
            plsc.store_scatter(dst_sc, [ridx], dvr, mask=lane < rem)
            plsc.store_scatter(zrow_v, [ridx], dvr >> 4, mask=lane < rem)

        @pl.loop(0, CHUNK // 4)
        def _grp(i):
            rows0 = row_off + 4 * i
            rows1 = row_off + 4 * i + 2
            sa0 = jnp.zeros((16,), jnp.float32)
            sb0 = jnp.zeros((16,), jnp.float32)
            sa1 = jnp.zeros((16,), jnp.float32)
            sb1 = jnp.zeros((16,), jnp.float32)
            for d in range(0, DH, 2):
                sa0 = sa0 + (plsc.load_gather(kbuf, [rows0, cols[d]])
                             * plsc.load_gather(qbuf, [rows0, cols[d]]))
                sa1 = sa1 + (plsc.load_gather(kbuf, [rows1, cols[d]])
                             * plsc.load_gather(qbuf, [rows1, cols[d]]))
                sb0 = sb0 + (plsc.load_gather(kbuf, [rows0, cols[d + 1]])
                             * plsc.load_gather(qbuf, [rows0, cols[d + 1]]))
                sb1 = sb1 + (plsc.load_gather(kbuf, [rows1, cols[d + 1]])
                             * plsc.load_gather(qbuf, [rows1, cols[d + 1]]))
            sval0 = jnp.exp(jnp.clip((sa0 + sb0) * 0.25, -5.0, 5.0))
            sval1 = jnp.exp(jnp.clip((sa1 + sb1) * 0.25, -5.0, 5.0))
            svbuf[pl.ds(i * 32, 16)] = sval0
            svbuf[pl.ds(i * 32 + 16, 16)] = sval1

        cv.wait()

        @pl.loop(0, CHUNK // 4)
        def _vph(i):
            rows0 = row_off + 4 * i
            rows1 = row_off + 4 * i + 2
            sval0 = svbuf[pl.ds(i * 32, 16)]
            sval1 = svbuf[pl.ds(i * 32 + 16, 16)]
            for d in range(DH):
                v0 = plsc.load_gather(vbuf, [rows0, cols[d]])
                v1 = plsc.load_gather(vbuf, [rows1, cols[d]])
                plsc.store_scatter(contrib, [rows0, cols[d]], v0 * sval0)
                plsc.store_scatter(contrib, [rows1, cols[d]], v1 * sval1)

        @pl.loop(0, CHUNK)
        def _zed(e):
            dsp = plsc.load_gather(dst_sc, [jnp.full((16,), e, jnp.int32)])
            for j in range(8):
                contrib_z[e, pl.ds(j * 16, 16)] = zero16
            sv = svbuf[pl.ds(e * 8, 16)]
            plsc.store_scatter(
                contrib_z,
                [jnp.full((16,), e, jnp.int32), (dsp & 15) * 8 + lane],
                sv, mask=lane < 8)

        cw = pltpu.async_copy(contrib, acc_wv.at[dst_sc], sem_s, add=True)
        cz = pltpu.async_copy(contrib_z, acc_z.at[zrow_v], sem_z, add=True)
        cw.wait()
        cz.wait()

    @pl.loop(0, n_super)
    def _sup(sp):
        sbase = ebase0 + sp * (SUP * CHUNK)
        pltpu.sync_copy(src_hbm.at[pl.ds(sbase, SUP * CHUNK)],
                        srcb.at[pl.ds(0, SUP * CHUNK)])
        pltpu.sync_copy(dst_hbm.at[pl.ds(sbase, SUP * CHUNK)],
                        dstb.at[pl.ds(0, SUP * CHUNK)])
        pend = issue_gathers(0, sbase)
        for sub in range(SUP):
            nxt = issue_gathers(sub + 1, sbase) if sub + 1 < SUP else None
            for d_ in pend:
                d_.wait()
            pend = nxt
            compute_chunk(sub)

    plsc.subcore_barrier()

    @pl.loop(0, rows_per_tile // IO_ROWS)
    def _out(j):
        r0 = s * rows_per_tile + j * IO_ROWS
        pltpu.sync_copy(acc_wv.at[pl.ds(r0, IO_ROWS)], io0)
        pltpu.sync_copy(io0, out_wv.at[c, pl.ds(r0, IO_ROWS)])

    zr0 = s * zrows_per_tile
    pltpu.sync_copy(acc_z.at[pl.ds(zr0, zrows_per_tile)],
                    iobuf.at[pl.ds(0, zrows_per_tile)])
    pltpu.sync_copy(iobuf.at[pl.ds(0, zrows_per_tile)],
                    out_z.at[c, pl.ds(zr0, zrows_per_tile)])


def _edge_aggregate(q, k, v, src, dst):
    fn = pl.kernel(
        _sc_edge_kernel,
        out_type=[
            jax.ShapeDtypeStruct((NC, N_PAD, D), jnp.float32),
            jax.ShapeDtypeStruct((NC, ZROWS, D), jnp.float32),
        ],
        mesh=plsc.VectorSubcoreMesh(core_axis_name="c", subcore_axis_name="s",
                                    num_cores=NC, num_subcores=NS),
        compiler_params=pltpu.CompilerParams(needs_layout_passes=False),
        scratch_types=[
            pltpu.VMEM((SUP * CHUNK + 16,), jnp.int32),
            pltpu.VMEM((SUP * CHUNK + 16,), jnp.int32),
            pltpu.VMEM((2 * CHUNK, D), jnp.float32),
            pltpu.VMEM((2 * CHUNK, D), jnp.float32),
            pltpu.VMEM((CHUNK, D), jnp.float32),
            pltpu.VMEM((CHUNK, D), jnp.float32),
            pltpu.VMEM((CHUNK, D), jnp.float32),
            pltpu.VMEM((CHUNK * 8 + 8,), jnp.float32),
            pltpu.VMEM((CHUNK,), jnp.int32),
            pltpu.VMEM((CHUNK,), jnp.int32),
            pltpu.VMEM_SHARED((N_PAD, D), jnp.float32),
            pltpu.VMEM_SHARED((ZROWS, D), jnp.float32),
            pltpu.SemaphoreType.DMA,
            pltpu.SemaphoreType.DMA,
            pltpu.SemaphoreType.DMA,
            pltpu.SemaphoreType.DMA,
            pltpu.SemaphoreType.DMA,
        ],
    )
    return fn(q, k, v, src, dst)


_BLK = 1000


def _qkv_body(h_ref, wqt_ref, wkt_ref, wvt_ref, q_ref, k_ref, v_ref):
    hb = h_ref[...]
    q_ref[...] = jnp.dot(hb, wqt_ref[...], preferred_element_type=jnp.float32)
    k_ref[...] = jnp.dot(hb, wkt_ref[...], preferred_element_type=jnp.float32)
    v_ref[...] = jnp.dot(hb, wvt_ref[...], preferred_element_type=jnp.float32)


def _qkv(h, wqt, wkt, wvt):
    n = h.shape[0]
    grid = n // _BLK
    return pl.pallas_call(
        _qkv_body,
        grid=(grid,),
        in_specs=[
            pl.BlockSpec((_BLK, D), lambda i: (i, 0)),
            pl.BlockSpec((D, D), lambda i: (0, 0)),
            pl.BlockSpec((D, D), lambda i: (0, 0)),
            pl.BlockSpec((D, D), lambda i: (0, 0)),
        ],
        out_specs=[
            pl.BlockSpec((_BLK, D), lambda i: (i, 0)),
            pl.BlockSpec((_BLK, D), lambda i: (i, 0)),
            pl.BlockSpec((_BLK, D), lambda i: (i, 0)),
        ],
        out_shape=[jax.ShapeDtypeStruct((n, D), jnp.float32)] * 3,
    )(h, wqt, wkt, wvt)


def _attn_out_body(wv_ref, z_ref, h_ref, r_ref, wot_ref, bo_ref, x1_ref,
                   st_ref):
    wv = wv_ref[0] + wv_ref[1]
    z = z_ref[0] + z_ref[1]
    zr = jnp.dot(z, r_ref[...], preferred_element_type=jnp.float32)
    head_out = wv / zr
    x = (jnp.dot(head_out, wot_ref[...], preferred_element_type=jnp.float32)
         + bo_ref[...] + h_ref[...])
    x1_ref[...] = x

    @pl.when(pl.program_id(0) == 0)
    def _():
        st_ref[...] = jnp.zeros_like(st_ref)

    st_ref[0:1, :] += jnp.sum(x, axis=0, keepdims=True)
    st_ref[1:2, :] += jnp.sum(x * x, axis=0, keepdims=True)


def _attn_out(acc_wv, zf, h, rmat, wot, bo2):
    n = h.shape[0]
    grid = n // _BLK
    return pl.pallas_call(
        _attn_out_body,
        grid=(grid,),
        in_specs=[
            pl.BlockSpec((NC, _BLK, D), lambda i: (0, i, 0)),
            pl.BlockSpec((NC, _BLK, H), lambda i: (0, i, 0)),
            pl.BlockSpec((_BLK, D), lambda i: (i, 0)),
            pl.BlockSpec((H, D), lambda i: (0, 0)),
            pl.BlockSpec((D, D), lambda i: (0, 0)),
            pl.BlockSpec((1, D), lambda i: (0, 0)),
        ],
        out_specs=[
            pl.BlockSpec((_BLK, D), lambda i: (i, 0)),
            pl.BlockSpec((8, D), lambda i: (0, 0)),
        ],
        out_shape=[
            jax.ShapeDtypeStruct((n, D), jnp.float32),
            jax.ShapeDtypeStruct((8, D), jnp.float32),
        ],
    )(acc_wv, zf, h, rmat, wot, bo2)


def _ffn_body(x1_ref, st_ref, g1_ref, be1_ref, w1t_ref, b1_ref, w2t_ref,
              b2_ref, x2_ref, st2_ref):
    inv_n = 1.0 / N_NODES
    mu = st_ref[0:1, :] * inv_n
    var = st_ref[1:2, :] * inv_n - mu * mu
    inv = lax.rsqrt(var + 1e-5)
    xn = (x1_ref[...] - mu) * (inv * g1_ref[...]) + be1_ref[...]
    t = jnp.maximum(
        jnp.dot(xn, w1t_ref[...], preferred_element_type=jnp.float32)
        + b1_ref[...], 0.0)
    y = jnp.dot(t, w2t_ref[...], preferred_element_type=jnp.float32) + b2_ref[...]
    x2 = xn + y
    x2_ref[...] = x2

    @pl.when(pl.program_id(0) == 0)
    def _():
        st2_ref[...] = jnp.zeros_like(st2_ref)

    st2_ref[0:1, :] += jnp.sum(x2, axis=0, keepdims=True)
    st2_ref[1:2, :] += jnp.sum(x2 * x2, axis=0, keepdims=True)


def _ffn(x1, st, g1b, be1b, w1t, b1b, w2t, b2b):
    n = x1.shape[0]
    grid = n // _BLK
    return pl.pallas_call(
        _ffn_body,
        grid=(grid,),
        in_specs=[
            pl.BlockSpec((_BLK, D), lambda i: (i, 0)),
            pl.BlockSpec((8, D), lambda i: (0, 0)),
            pl.BlockSpec((1, D), lambda i: (0, 0)),
            pl.BlockSpec((1, D), lambda i: (0, 0)),
            pl.BlockSpec((D, 2 * D), lambda i: (0, 0)),
            pl.BlockSpec((1, 2 * D), lambda i: (0, 0)),
            pl.BlockSpec((2 * D, D), lambda i: (0, 0)),
            pl.BlockSpec((1, D), lambda i: (0, 0)),
        ],
        out_specs=[
            pl.BlockSpec((_BLK, D), lambda i: (i, 0)),
            pl.BlockSpec((8, D), lambda i: (0, 0)),
        ],
        out_shape=[
            jax.ShapeDtypeStruct((n, D), jnp.float32),
            jax.ShapeDtypeStruct((8, D), jnp.float32),
        ],
    )(x1, st, g1b, be1b, w1t, b1b, w2t, b2b)


def _bn2_body(x2_ref, st2_ref, g2_ref, be2_ref, out_ref):
    inv_n = 1.0 / N_NODES
    mu = st2_ref[0:1, :] * inv_n
    var = st2_ref[1:2, :] * inv_n - mu * mu
    inv = lax.rsqrt(var + 1e-5)
    out_ref[...] = (x2_ref[...] - mu) * (inv * g2_ref[...]) + be2_ref[...]


def _bn2(x2, st2, g2b, be2b):
    n = x2.shape[0]
    grid = n // _BLK
    return pl.pallas_call(
        _bn2_body,
        grid=(grid,),
        in_specs=[
            pl.BlockSpec((_BLK, D), lambda i: (i, 0)),
            pl.BlockSpec((8, D), lambda i: (0, 0)),
            pl.BlockSpec((1, D), lambda i: (0, 0)),
            pl.BlockSpec((1, D), lambda i: (0, 0)),
        ],
        out_specs=pl.BlockSpec((_BLK, D), lambda i: (i, 0)),
        out_shape=jax.ShapeDtypeStruct((n, D), jnp.float32),
    )(x2, st2, g2b, be2b)


def kernel(h, edge_index, Wq, Wk, Wv, Wo, bo, W1, b1, W2, b2, g1, be1, g2, be2):
    cp = jnp.arange(D, dtype=jnp.int32)
    perm_orig = (cp % H) * DH + cp // H
    wqt = Wq.T[:, perm_orig]
    wkt = Wk.T[:, perm_orig]
    wvt = Wv.T[:, perm_orig]
    wot = Wo.T[perm_orig, :]
    w1t = W1.T
    w2t = W2.T
    bo2 = bo.reshape(1, D)
    b1b = b1.reshape(1, 2 * D)
    b2b = b2.reshape(1, D)
    g1b = g1.reshape(1, D)
    be1b = be1.reshape(1, D)
    g2b = g2.reshape(1, D)
    be2b = be2.reshape(1, D)
    rmat = (cp[None, :] % H
            == jnp.arange(H, dtype=jnp.int32)[:, None]).astype(jnp.float32)

    q, k, v = _qkv(h, wqt, wkt, wvt)
    acc_wv, acc_z = _edge_aggregate(q, k, v, edge_index[0], edge_index[1])
    zf = acc_z.reshape(NC, N_PAD, H)
    x1, st = _attn_out(acc_wv, zf, h, rmat, wot, bo2)
    x2, st2 = _ffn(x1, st, g1b, be1b, w1t, b1b, w2t, b2b)
    return _bn2(x2, st2, g2b, be2b)


# speedup vs baseline: 1.9979x; 1.0305x over previous
import functools

import jax
import jax.numpy as jnp
from jax import lax
from jax.experimental import pallas as pl
from jax.experimental.pallas import tpu as pltpu
from jax.experimental.pallas import tpu_sc as plsc

N_NODES = 10000
N_PAD = 10240
D = 128
H = 8
DH = 16

NC = 2
NS = 16
NW = NC * NS

ZROWS = N_PAD // 16

CHUNK = 40
SUP = 10
IO_ROWS = 40


def _sc_edge_kernel(q_hbm, k_hbm, v_hbm, src_hbm, dst_hbm,
                    out_wv, out_z,
                    srcb, dstb, kqv0, kqv1, vbuf, contrib, contrib_z,
                    svbuf, dst_sc, zrow_v, acc_wv, acc_z,
                    sem_g0, sem_g1, sem_v, sem_s, sem_z):
    c = lax.axis_index("c")
    s = lax.axis_index("s")
    w = s * NC + c

    e_total = src_hbm.shape[0]
    e_per_tile = e_total // NW
    n_super = e_per_tile // (SUP * CHUNK)
    rows_per_tile = N_PAD // NS
    zrows_per_tile = ZROWS // NS

    lane = lax.iota(jnp.int32, 16)
    zero16 = jnp.zeros((16,), jnp.float32)
    kq = (kqv0, kqv1)
    sem_g = (sem_g0, sem_g1)
    iobuf = kqv0

    @pl.loop(0, IO_ROWS * 8)
    def _z(i):
        iobuf[i // 8, pl.ds((i % 8) * 16, 16)] = zero16

    @pl.loop(0, CHUNK * 8)
    def _zc(i):
        contrib_z[i // 8, pl.ds((i % 8) * 16, 16)] = zero16

    io0 = iobuf.at[pl.ds(0, IO_ROWS)]

    @pl.loop(0, rows_per_tile // IO_ROWS)
    def _z2(j):
        pltpu.sync_copy(io0, acc_wv.at[pl.ds(s * rows_per_tile + j * IO_ROWS, IO_ROWS)])

    pltpu.sync_copy(iobuf.at[pl.ds(0, zrows_per_tile)],
                    acc_z.at[pl.ds(s * zrows_per_tile, zrows_per_tile)])

    plsc.subcore_barrier()

    row_off = lane // H
    head = lane % H
    cols = tuple(head + ((d + row_off) % DH) * H for d in range(DH))
    ebase0 = w * e_per_tile

    def issue_gathers(sub, sbase):
        b = sub % 2
        buf = kq[b]
        sem = sem_g[b]
        so = sub * CHUNK
        ck = pltpu.async_copy(k_hbm.at[srcb.at[pl.ds(so, CHUNK)]],
                              buf.at[pl.ds(0, CHUNK)], sem)
        cq = pltpu.async_copy(q_hbm.at[dstb.at[pl.ds(so, CHUNK)]],
                              buf.at[pl.ds(CHUNK, CHUNK)], sem)
        return (ck, cq)

    def compute_chunk(sub):
        b = sub % 2
        buf = kq[b]
        kbuf = buf.at[pl.ds(0, CHUNK)]
        qbuf = buf.at[pl.ds(CHUNK, CHUNK)]
        so = sub * CHUNK
        cv = pltpu.async_copy(v_hbm.at[srcb.at[pl.ds(so, CHUNK)]],
                              vbuf, sem_v)
        for t in range(CHUNK // 16):
            dv = dstb[pl.ds(so + t * 16, 16)]
            dst_sc[pl.ds(t * 16, 16)] = dv
            zrow_v[pl.ds(t * 16, 16)] = dv >> 4
        rem = CHUNK % 16
        if rem:
            ridx = jnp.minimum(lane + (CHUNK // 16) * 16, CHUNK - 1)
            dvr = plsc.load_gather(dstb, [so + ridx])
            plsc.store_scatter(dst_sc, [ridx], dvr, mask=lane < rem)
            plsc.store_scatter(zrow_v, [ridx], dvr >> 4, mask=lane < rem)

        @pl.loop(0, CHUNK // 4)
        def _grp(i):
            rows0 = row_off + 4 * i
            rows1 = row_off + 4 * i + 2
            sa0 = jnp.zeros((16,), jnp.float32)
            sb0 = jnp.zeros((16,), jnp.float32)
            sa1 = jnp.zeros((16,), jnp.float32)
            sb1 = jnp.zeros((16,), jnp.float32)
            for d in range(0, DH, 2):
                sa0 = sa0 + (plsc.load_gather(kbuf, [rows0, cols[d]])
                             * plsc.load_gather(qbuf, [rows0, cols[d]]))
                sa1 = sa1 + (plsc.load_gather(kbuf, [rows1, cols[d]])
                             * plsc.load_gather(qbuf, [rows1, cols[d]]))
                sb0 = sb0 + (plsc.load_gather(kbuf, [rows0, cols[d + 1]])
                             * plsc.load_gather(qbuf, [rows0, cols[d + 1]]))
                sb1 = sb1 + (plsc.load_gather(kbuf, [rows1, cols[d + 1]])
                             * plsc.load_gather(qbuf, [rows1, cols[d + 1]]))
            sval0 = jnp.exp(jnp.clip((sa0 + sb0) * 0.25, -5.0, 5.0))
            sval1 = jnp.exp(jnp.clip((sa1 + sb1) * 0.25, -5.0, 5.0))
            svbuf[pl.ds(i * 32, 16)] = sval0
            svbuf[pl.ds(i * 32 + 16, 16)] = sval1

        cv.wait()

        @pl.loop(0, CHUNK // 4)
        def _vph(i):
            rows0 = row_off + 4 * i
            rows1 = row_off + 4 * i + 2
            sval0 = svbuf[pl.ds(i * 32, 16)]
            sval1 = svbuf[pl.ds(i * 32 + 16, 16)]
            for d in range(DH):
                v0 = plsc.load_gather(vbuf, [rows0, cols[d]])
                v1 = plsc.load_gather(vbuf, [rows1, cols[d]])
                plsc.store_scatter(contrib, [rows0, cols[d]], v0 * sval0)
                plsc.store_scatter(contrib, [rows1, cols[d]], v1 * sval1)

        @pl.loop(0, CHUNK)
        def _zed(e):
            dsp = plsc.load_gather(dst_sc, [jnp.full((16,), e, jnp.int32)])
            for j in range(8):
                contrib_z[e, pl.ds(j * 16, 16)] = zero16
            sv = svbuf[pl.ds(e * 8, 16)]
            plsc.store_scatter(
                contrib_z,
                [jnp.full((16,), e, jnp.int32), (dsp & 15) * 8 + lane],
                sv, mask=lane < 8)

        cw = pltpu.async_copy(contrib, acc_wv.at[dst_sc], sem_s, add=True)
        cz = pltpu.async_copy(contrib_z, acc_z.at[zrow_v], sem_z, add=True)
        cw.wait()
        cz.wait()

    @pl.loop(0, n_super)
    def _sup(sp):
        sbase = ebase0 + sp * (SUP * CHUNK)
        pltpu.sync_copy(src_hbm.at[pl.ds(sbase, SUP * CHUNK)],
                        srcb.at[pl.ds(0, SUP * CHUNK)])
        pltpu.sync_copy(dst_hbm.at[pl.ds(sbase, SUP * CHUNK)],
                        dstb.at[pl.ds(0, SUP * CHUNK)])
        pend = issue_gathers(0, sbase)
        for sub in range(SUP):
            nxt = issue_gathers(sub + 1, sbase) if sub + 1 < SUP else None
            for d_ in pend:
                d_.wait()
            pend = nxt
            compute_chunk(sub)

    plsc.subcore_barrier()

    @pl.loop(0, rows_per_tile // IO_ROWS)
    def _out(j):
        r0 = s * rows_per_tile + j * IO_ROWS
        pltpu.sync_copy(acc_wv.at[pl.ds(r0, IO_ROWS)], io0)
        pltpu.sync_copy(io0, out_wv.at[c, pl.ds(r0, IO_ROWS)])

    zr0 = s * zrows_per_tile
    pltpu.sync_copy(acc_z.at[pl.ds(zr0, zrows_per_tile)],
                    iobuf.at[pl.ds(0, zrows_per_tile)])
    pltpu.sync_copy(iobuf.at[pl.ds(0, zrows_per_tile)],
                    out_z.at[c, pl.ds(zr0, zrows_per_tile)])


def _edge_aggregate(q, k, v, src, dst):
    fn = pl.kernel(
        _sc_edge_kernel,
        out_type=[
            jax.ShapeDtypeStruct((NC, N_PAD, D), jnp.float32),
            jax.ShapeDtypeStruct((NC, ZROWS, D), jnp.float32),
        ],
        mesh=plsc.VectorSubcoreMesh(core_axis_name="c", subcore_axis_name="s",
                                    num_cores=NC, num_subcores=NS),
        compiler_params=pltpu.CompilerParams(needs_layout_passes=False),
        scratch_types=[
            pltpu.VMEM((SUP * CHUNK + 16,), jnp.int32),
            pltpu.VMEM((SUP * CHUNK + 16,), jnp.int32),
            pltpu.VMEM((2 * CHUNK, D), jnp.float32),
            pltpu.VMEM((2 * CHUNK, D), jnp.float32),
            pltpu.VMEM((CHUNK, D), jnp.float32),
            pltpu.VMEM((CHUNK, D), jnp.float32),
            pltpu.VMEM((CHUNK, D), jnp.float32),
            pltpu.VMEM((CHUNK * 8 + 8,), jnp.float32),
            pltpu.VMEM((CHUNK,), jnp.int32),
            pltpu.VMEM((CHUNK,), jnp.int32),
            pltpu.VMEM_SHARED((N_PAD, D), jnp.float32),
            pltpu.VMEM_SHARED((ZROWS, D), jnp.float32),
            pltpu.SemaphoreType.DMA,
            pltpu.SemaphoreType.DMA,
            pltpu.SemaphoreType.DMA,
            pltpu.SemaphoreType.DMA,
            pltpu.SemaphoreType.DMA,
        ],
    )
    return fn(q, k, v, src, dst)


_BLK = 1000


def _qkv_body(h_ref, wqt_ref, wkt_ref, wvt_ref, q_ref, k_ref, v_ref):
    hb = h_ref[...]
    q_ref[...] = jnp.dot(hb, wqt_ref[...], preferred_element_type=jnp.float32)
    k_ref[...] = jnp.dot(hb, wkt_ref[...], preferred_element_type=jnp.float32)
    v_ref[...] = jnp.dot(hb, wvt_ref[...], preferred_element_type=jnp.float32)


def _qkv(h, wqt, wkt, wvt):
    n = h.shape[0]
    grid = n // _BLK
    return pl.pallas_call(
        _qkv_body,
        grid=(grid,),
        in_specs=[
            pl.BlockSpec((_BLK, D), lambda i: (i, 0)),
            pl.BlockSpec((D, D), lambda i: (0, 0)),
            pl.BlockSpec((D, D), lambda i: (0, 0)),
            pl.BlockSpec((D, D), lambda i: (0, 0)),
        ],
        out_specs=[
            pl.BlockSpec((_BLK, D), lambda i: (i, 0)),
            pl.BlockSpec((_BLK, D), lambda i: (i, 0)),
            pl.BlockSpec((_BLK, D), lambda i: (i, 0)),
        ],
        out_shape=[jax.ShapeDtypeStruct((n, D), jnp.float32)] * 3,
    )(h, wqt, wkt, wvt)


def _attn_out_body(wv_ref, z_ref, h_ref, r_ref, wot_ref, bo_ref, x1_ref,
                   st_ref):
    wv = wv_ref[0] + wv_ref[1]
    z = z_ref[0] + z_ref[1]
    zr = jnp.dot(z, r_ref[...], preferred_element_type=jnp.float32)
    head_out = wv / zr
    x = (jnp.dot(head_out, wot_ref[...], preferred_element_type=jnp.float32)
         + bo_ref[...] + h_ref[...])
    x1_ref[...] = x

    @pl.when(pl.program_id(0) == 0)
    def _():
        st_ref[...] = jnp.zeros_like(st_ref)

    st_ref[0:1, :] += jnp.sum(x, axis=0, keepdims=True)
    st_ref[1:2, :] += jnp.sum(x * x, axis=0, keepdims=True)


def _attn_out(acc_wv, zf, h, rmat, wot, bo2):
    n = h.shape[0]
    grid = n // _BLK
    return pl.pallas_call(
        _attn_out_body,
        grid=(grid,),
        in_specs=[
            pl.BlockSpec((NC, _BLK, D), lambda i: (0, i, 0)),
            pl.BlockSpec((NC, _BLK, H), lambda i: (0, i, 0)),
            pl.BlockSpec((_BLK, D), lambda i: (i, 0)),
            pl.BlockSpec((H, D), lambda i: (0, 0)),
            pl.BlockSpec((D, D), lambda i: (0, 0)),
            pl.BlockSpec((1, D), lambda i: (0, 0)),
        ],
        out_specs=[
            pl.BlockSpec((_BLK, D), lambda i: (i, 0)),
            pl.BlockSpec((8, D), lambda i: (0, 0)),
        ],
        out_shape=[
            jax.ShapeDtypeStruct((n, D), jnp.float32),
            jax.ShapeDtypeStruct((8, D), jnp.float32),
        ],
    )(acc_wv, zf, h, rmat, wot, bo2)


def _ffn_body(x1_ref, st_ref, g1_ref, be1_ref, w1t_ref, b1_ref, w2t_ref,
              b2_ref, x2_ref, st2_ref):
    inv_n = 1.0 / N_NODES
    mu = st_ref[0:1, :] * inv_n
    var = st_ref[1:2, :] * inv_n - mu * mu
    inv = lax.rsqrt(var + 1e-5)
    xn = (x1_ref[...] - mu) * (inv * g1_ref[...]) + be1_ref[...]
    t = jnp.maximum(
        jnp.dot(xn, w1t_ref[...], preferred_element_type=jnp.float32)
        + b1_ref[...], 0.0)
    y = jnp.dot(t, w2t_ref[...], preferred_element_type=jnp.float32) + b2_ref[...]
    x2 = xn + y
    x2_ref[...] = x2

    @pl.when(pl.program_id(0) == 0)
    def _():
        st2_ref[...] = jnp.zeros_like(st2_ref)

    st2_ref[0:1, :] += jnp.sum(x2, axis=0, keepdims=True)
    st2_ref[1:2, :] += jnp.sum(x2 * x2, axis=0, keepdims=True)


def _ffn(x1, st, g1b, be1b, w1t, b1b, w2t, b2b):
    n = x1.shape[0]
    grid = n // _BLK
    return pl.pallas_call(
        _ffn_body,
        grid=(grid,),
        in_specs=[
            pl.BlockSpec((_BLK, D), lambda i: (i, 0)),
            pl.BlockSpec((8, D), lambda i: (0, 0)),
            pl.BlockSpec((1, D), lambda i: (0, 0)),
            pl.BlockSpec((1, D), lambda i: (0, 0)),
            pl.BlockSpec((D, 2 * D), lambda i: (0, 0)),
            pl.BlockSpec((1, 2 * D), lambda i: (0, 0)),
            pl.BlockSpec((2 * D, D), lambda i: (0, 0)),
            pl.BlockSpec((1, D), lambda i: (0, 0)),
        ],
        out_specs=[
            pl.BlockSpec((_BLK, D), lambda i: (i, 0)),
            pl.BlockSpec((8, D), lambda i: (0, 0)),
        ],
        out_shape=[
            jax.ShapeDtypeStruct((n, D), jnp.float32),
            jax.ShapeDtypeStruct((8, D), jnp.float32),
        ],
    )(x1, st, g1b, be1b, w1t, b1b, w2t, b2b)


def _bn2_body(x2_ref, st2_ref, g2_ref, be2_ref, out_ref):
    inv_n = 1.0 / N_NODES
    mu = st2_ref[0:1, :] * inv_n
    var = st2_ref[1:2, :] * inv_n - mu * mu
    inv = lax.rsqrt(var + 1e-5)
    out_ref[...] = (x2_ref[...] - mu) * (inv * g2_ref[...]) + be2_ref[...]


def _bn2(x2, st2, g2b, be2b):
    n = x2.shape[0]
    grid = n // _BLK
    return pl.pallas_call(
        _bn2_body,
        grid=(grid,),
        in_specs=[
            pl.BlockSpec((_BLK, D), lambda i: (i, 0)),
            pl.BlockSpec((8, D), lambda i: (0, 0)),
            pl.BlockSpec((1, D), lambda i: (0, 0)),
            pl.BlockSpec((1, D), lambda i: (0, 0)),
        ],
        out_specs=pl.BlockSpec((_BLK, D), lambda i: (i, 0)),
        out_shape=jax.ShapeDtypeStruct((n, D), jnp.float32),
    )(x2, st2, g2b, be2b)


def kernel(h, edge_index, Wq, Wk, Wv, Wo, bo, W1, b1, W2, b2, g1, be1, g2, be2):
    cp = jnp.arange(D, dtype=jnp.int32)
    perm_orig = (cp % H) * DH + cp // H
    wqt = Wq.T[:, perm_orig]
    wkt = Wk.T[:, perm_orig]
    wvt = Wv.T[:, perm_orig]
    wot = Wo.T[perm_orig, :]
    w1t = W1.T
    w2t = W2.T
    bo2 = bo.reshape(1, D)
    b1b = b1.reshape(1, 2 * D)
    b2b = b2.reshape(1, D)
    g1b = g1.reshape(1, D)
    be1b = be1.reshape(1, D)
    g2b = g2.reshape(1, D)
    be2b = be2.reshape(1, D)
    rmat = (cp[None, :] % H
            == jnp.arange(H, dtype=jnp.int32)[:, None]).astype(jnp.float32)

    q, k, v = _qkv(h, wqt, wkt, wvt)
    acc_wv, acc_z = _edge_aggregate(q, k, v, edge_index[0], edge_index[1])
    zf = acc_z.reshape(NC, N_PAD, H)
    x1, st = _attn_out(acc_wv, zf, h, rmat, wot, bo2)
    x2, st2 = _ffn(x1, st, g1b, be1b, w1t, b1b, w2t, b2b)
    return _bn2(x2, st2, g2b, be2b)


# speedup vs baseline: 2.2157x; 1.1090x over previous
import functools

import jax
import jax.numpy as jnp
from jax import lax
from jax.experimental import pallas as pl
from jax.experimental.pallas import tpu as pltpu
from jax.experimental.pallas import tpu_sc as plsc

N_NODES = 10000
N_PAD = 10240
D = 128
H = 8
DH = 16

NC = 2
NS = 16
NW = NC * NS

ZROWS = N_PAD // 16

CHUNK = 40
SUP = 10
IO_ROWS = 40


def _sc_edge_kernel(q_hbm, k_hbm, v_hbm, src_hbm, dst_hbm,
                    out_wv, out_z,
                    srcb, dstb, kqv0, kqv1, vbuf, contrib, contrib_z,
                    svbuf, dst_sc0, dst_sc1, zrow0, zrow1, acc_wv, acc_z,
                    sem_g0, sem_g1, sem_v, sem_s, sem_z, sem_zf):
    c = lax.axis_index("c")
    s = lax.axis_index("s")
    w = s * NC + c

    e_total = src_hbm.shape[0]
    e_per_tile = e_total // NW
    n_super = e_per_tile // (SUP * CHUNK)
    rows_per_tile = N_PAD // NS
    zrows_per_tile = ZROWS // NS

    lane = lax.iota(jnp.int32, 16)
    zero16 = jnp.zeros((16,), jnp.float32)
    kq = (kqv0, kqv1)
    sem_g = (sem_g0, sem_g1)
    dst_scb = (dst_sc0, dst_sc1)
    zrowb = (zrow0, zrow1)
    iobuf = kqv0

    @pl.loop(0, IO_ROWS * 8)
    def _z(i):
        iobuf[i // 8, pl.ds((i % 8) * 16, 16)] = zero16

    @pl.loop(0, CHUNK * 8)
    def _zc(i):
        contrib_z[i // 8, pl.ds((i % 8) * 16, 16)] = zero16

    io0 = iobuf.at[pl.ds(0, IO_ROWS)]

    @pl.loop(0, rows_per_tile // IO_ROWS)
    def _z2(j):
        pltpu.sync_copy(io0, acc_wv.at[pl.ds(s * rows_per_tile + j * IO_ROWS, IO_ROWS)])

    pltpu.sync_copy(iobuf.at[pl.ds(0, zrows_per_tile)],
                    acc_z.at[pl.ds(s * zrows_per_tile, zrows_per_tile)])

    plsc.subcore_barrier()

    row_off = lane // H
    head = lane % H
    cols = tuple(head + ((d + row_off) % DH) * H for d in range(DH))
    ebase0 = w * e_per_tile

    def issue_gathers(sub, sbase):
        b = sub % 2
        buf = kq[b]
        sem = sem_g[b]
        so = sub * CHUNK
        ck = pltpu.async_copy(k_hbm.at[srcb.at[pl.ds(so, CHUNK)]],
                              buf.at[pl.ds(0, CHUNK)], sem)
        cq = pltpu.async_copy(q_hbm.at[dstb.at[pl.ds(so, CHUNK)]],
                              buf.at[pl.ds(CHUNK, CHUNK)], sem)
        return (ck, cq)

    def compute_chunk(sub, pend_sc):
        b = sub % 2
        buf = kq[b]
        kbuf = buf.at[pl.ds(0, CHUNK)]
        qbuf = buf.at[pl.ds(CHUNK, CHUNK)]
        dst_sc = dst_scb[b]
        zrow_v = zrowb[b]
        so = sub * CHUNK
        cv = pltpu.async_copy(v_hbm.at[srcb.at[pl.ds(so, CHUNK)]],
                              vbuf, sem_v)
        for t in range(CHUNK // 16):
            dv = dstb[pl.ds(so + t * 16, 16)]
            dst_sc[pl.ds(t * 16, 16)] = dv
            zrow_v[pl.ds(t * 16, 16)] = dv >> 4
        rem = CHUNK % 16
        if rem:
            ridx = jnp.minimum(lane + (CHUNK // 16) * 16, CHUNK - 1)
            dvr = plsc.load_gather(dstb, [so + ridx])
            plsc.store_scatter(dst_sc, [ridx], dvr, mask=lane < rem)
            plsc.store_scatter(zrow_v, [ridx], dvr >> 4, mask=lane < rem)

        @pl.loop(0, CHUNK // 4)
        def _grp(i):
            rows0 = row_off + 4 * i
            rows1 = row_off + 4 * i + 2
            sa0 = jnp.zeros((16,), jnp.float32)
            sb0 = jnp.zeros((16,), jnp.float32)
            sa1 = jnp.zeros((16,), jnp.float32)
            sb1 = jnp.zeros((16,), jnp.float32)
            for d in range(0, DH, 2):
                sa0 = sa0 + (plsc.load_gather(kbuf, [rows0, cols[d]])
                             * plsc.load_gather(qbuf, [rows0, cols[d]]))
                sa1 = sa1 + (plsc.load_gather(kbuf, [rows1, cols[d]])
                             * plsc.load_gather(qbuf, [rows1, cols[d]]))
                sb0 = sb0 + (plsc.load_gather(kbuf, [rows0, cols[d + 1]])
                             * plsc.load_gather(qbuf, [rows0, cols[d + 1]]))
                sb1 = sb1 + (plsc.load_gather(kbuf, [rows1, cols[d + 1]])
                             * plsc.load_gather(qbuf, [rows1, cols[d + 1]]))
            sval0 = jnp.exp(jnp.clip((sa0 + sb0) * 0.25, -5.0, 5.0))
            sval1 = jnp.exp(jnp.clip((sa1 + sb1) * 0.25, -5.0, 5.0))
            svbuf[pl.ds(i * 32, 16)] = sval0
            svbuf[pl.ds(i * 32 + 16, 16)] = sval1

        cv.wait()
        for d_ in pend_sc:
            d_.wait()
        zf = pltpu.async_copy(acc_wv.at[pl.ds(N_PAD - CHUNK, CHUNK)],
                              contrib_z, sem_zf)

        @pl.loop(0, CHUNK // 4)
        def _vph(i):
            rows0 = row_off + 4 * i
            rows1 = row_off + 4 * i + 2
            sval0 = svbuf[pl.ds(i * 32, 16)]
            sval1 = svbuf[pl.ds(i * 32 + 16, 16)]
            for d in range(DH):
                v0 = plsc.load_gather(vbuf, [rows0, cols[d]])
                v1 = plsc.load_gather(vbuf, [rows1, cols[d]])
                plsc.store_scatter(contrib, [rows0, cols[d]], v0 * sval0)
                plsc.store_scatter(contrib, [rows1, cols[d]], v1 * sval1)

        zf.wait()

        @pl.loop(0, CHUNK)
        def _zed(e):
            dsp = plsc.load_gather(dst_sc, [jnp.full((16,), e, jnp.int32)])
            sv = svbuf[pl.ds(e * 8, 16)]
            plsc.store_scatter(
                contrib_z,
                [jnp.full((16,), e, jnp.int32), (dsp & 15) * 8 + lane],
                sv, mask=lane < 8)

        cw = pltpu.async_copy(contrib, acc_wv.at[dst_sc], sem_s, add=True)
        cz = pltpu.async_copy(contrib_z, acc_z.at[zrow_v], sem_z, add=True)
        return (cw, cz)

    @pl.loop(0, n_super)
    def _sup(sp):
        sbase = ebase0 + sp * (SUP * CHUNK)
        pltpu.sync_copy(src_hbm.at[pl.ds(sbase, SUP * CHUNK)],
                        srcb.at[pl.ds(0, SUP * CHUNK)])
        pltpu.sync_copy(dst_hbm.at[pl.ds(sbase, SUP * CHUNK)],
                        dstb.at[pl.ds(0, SUP * CHUNK)])
        pend = issue_gathers(0, sbase)
        psc = ()
        for sub in range(SUP):
            nxt = issue_gathers(sub + 1, sbase) if sub + 1 < SUP else None
            for d_ in pend:
                d_.wait()
            pend = nxt
            psc = compute_chunk(sub, psc)
        for d_ in psc:
            d_.wait()

    plsc.subcore_barrier()

    @pl.loop(0, rows_per_tile // IO_ROWS)
    def _out(j):
        r0 = s * rows_per_tile + j * IO_ROWS
        pltpu.sync_copy(acc_wv.at[pl.ds(r0, IO_ROWS)], io0)
        pltpu.sync_copy(io0, out_wv.at[c, pl.ds(r0, IO_ROWS)])

    zr0 = s * zrows_per_tile
    pltpu.sync_copy(acc_z.at[pl.ds(zr0, zrows_per_tile)],
                    iobuf.at[pl.ds(0, zrows_per_tile)])
    pltpu.sync_copy(iobuf.at[pl.ds(0, zrows_per_tile)],
                    out_z.at[c, pl.ds(zr0, zrows_per_tile)])


def _edge_aggregate(q, k, v, src, dst):
    fn = pl.kernel(
        _sc_edge_kernel,
        out_type=[
            jax.ShapeDtypeStruct((NC, N_PAD, D), jnp.float32),
            jax.ShapeDtypeStruct((NC, ZROWS, D), jnp.float32),
        ],
        mesh=plsc.VectorSubcoreMesh(core_axis_name="c", subcore_axis_name="s",
                                    num_cores=NC, num_subcores=NS),
        compiler_params=pltpu.CompilerParams(needs_layout_passes=False),
        scratch_types=[
            pltpu.VMEM((SUP * CHUNK + 16,), jnp.int32),
            pltpu.VMEM((SUP * CHUNK + 16,), jnp.int32),
            pltpu.VMEM((2 * CHUNK, D), jnp.float32),
            pltpu.VMEM((2 * CHUNK, D), jnp.float32),
            pltpu.VMEM((CHUNK, D), jnp.float32),
            pltpu.VMEM((CHUNK, D), jnp.float32),
            pltpu.VMEM((CHUNK, D), jnp.float32),
            pltpu.VMEM((CHUNK * 8 + 8,), jnp.float32),
            pltpu.VMEM((CHUNK,), jnp.int32),
            pltpu.VMEM((CHUNK,), jnp.int32),
            pltpu.VMEM((CHUNK,), jnp.int32),
            pltpu.VMEM((CHUNK,), jnp.int32),
            pltpu.VMEM_SHARED((N_PAD, D), jnp.float32),
            pltpu.VMEM_SHARED((ZROWS, D), jnp.float32),
            pltpu.SemaphoreType.DMA,
            pltpu.SemaphoreType.DMA,
            pltpu.SemaphoreType.DMA,
            pltpu.SemaphoreType.DMA,
            pltpu.SemaphoreType.DMA,
            pltpu.SemaphoreType.DMA,
        ],
    )
    return fn(q, k, v, src, dst)


_BLK = 1000


def _qkv_body(h_ref, wqt_ref, wkt_ref, wvt_ref, q_ref, k_ref, v_ref):
    hb = h_ref[...]
    q_ref[...] = jnp.dot(hb, wqt_ref[...], preferred_element_type=jnp.float32)
    k_ref[...] = jnp.dot(hb, wkt_ref[...], preferred_element_type=jnp.float32)
    v_ref[...] = jnp.dot(hb, wvt_ref[...], preferred_element_type=jnp.float32)


def _qkv(h, wqt, wkt, wvt):
    n = h.shape[0]
    grid = n // _BLK
    return pl.pallas_call(
        _qkv_body,
        grid=(grid,),
        in_specs=[
            pl.BlockSpec((_BLK, D), lambda i: (i, 0)),
            pl.BlockSpec((D, D), lambda i: (0, 0)),
            pl.BlockSpec((D, D), lambda i: (0, 0)),
            pl.BlockSpec((D, D), lambda i: (0, 0)),
        ],
        out_specs=[
            pl.BlockSpec((_BLK, D), lambda i: (i, 0)),
            pl.BlockSpec((_BLK, D), lambda i: (i, 0)),
            pl.BlockSpec((_BLK, D), lambda i: (i, 0)),
        ],
        out_shape=[jax.ShapeDtypeStruct((n, D), jnp.float32)] * 3,
    )(h, wqt, wkt, wvt)


def _attn_out_body(wv_ref, z_ref, h_ref, r_ref, wot_ref, bo_ref, x1_ref,
                   st_ref):
    wv = wv_ref[0] + wv_ref[1]
    z = z_ref[0] + z_ref[1]
    zr = jnp.dot(z, r_ref[...], preferred_element_type=jnp.float32)
    head_out = wv / zr
    x = (jnp.dot(head_out, wot_ref[...], preferred_element_type=jnp.float32)
         + bo_ref[...] + h_ref[...])
    x1_ref[...] = x

    @pl.when(pl.program_id(0) == 0)
    def _():
        st_ref[...] = jnp.zeros_like(st_ref)

    st_ref[0:1, :] += jnp.sum(x, axis=0, keepdims=True)
    st_ref[1:2, :] += jnp.sum(x * x, axis=0, keepdims=True)


def _attn_out(acc_wv, zf, h, rmat, wot, bo2):
    n = h.shape[0]
    grid = n // _BLK
    return pl.pallas_call(
        _attn_out_body,
        grid=(grid,),
        in_specs=[
            pl.BlockSpec((NC, _BLK, D), lambda i: (0, i, 0)),
            pl.BlockSpec((NC, _BLK, H), lambda i: (0, i, 0)),
            pl.BlockSpec((_BLK, D), lambda i: (i, 0)),
            pl.BlockSpec((H, D), lambda i: (0, 0)),
            pl.BlockSpec((D, D), lambda i: (0, 0)),
            pl.BlockSpec((1, D), lambda i: (0, 0)),
        ],
        out_specs=[
            pl.BlockSpec((_BLK, D), lambda i: (i, 0)),
            pl.BlockSpec((8, D), lambda i: (0, 0)),
        ],
        out_shape=[
            jax.ShapeDtypeStruct((n, D), jnp.float32),
            jax.ShapeDtypeStruct((8, D), jnp.float32),
        ],
    )(acc_wv, zf, h, rmat, wot, bo2)


def _ffn_body(x1_ref, st_ref, g1_ref, be1_ref, w1t_ref, b1_ref, w2t_ref,
              b2_ref, x2_ref, st2_ref):
    inv_n = 1.0 / N_NODES
    mu = st_ref[0:1, :] * inv_n
    var = st_ref[1:2, :] * inv_n - mu * mu
    inv = lax.rsqrt(var + 1e-5)
    xn = (x1_ref[...] - mu) * (inv * g1_ref[...]) + be1_ref[...]
    t = jnp.maximum(
        jnp.dot(xn, w1t_ref[...], preferred_element_type=jnp.float32)
        + b1_ref[...], 0.0)
    y = jnp.dot(t, w2t_ref[...], preferred_element_type=jnp.float32) + b2_ref[...]
    x2 = xn + y
    x2_ref[...] = x2

    @pl.when(pl.program_id(0) == 0)
    def _():
        st2_ref[...] = jnp.zeros_like(st2_ref)

    st2_ref[0:1, :] += jnp.sum(x2, axis=0, keepdims=True)
    st2_ref[1:2, :] += jnp.sum(x2 * x2, axis=0, keepdims=True)


def _ffn(x1, st, g1b, be1b, w1t, b1b, w2t, b2b):
    n = x1.shape[0]
    grid = n // _BLK
    return pl.pallas_call(
        _ffn_body,
        grid=(grid,),
        in_specs=[
            pl.BlockSpec((_BLK, D), lambda i: (i, 0)),
            pl.BlockSpec((8, D), lambda i: (0, 0)),
            pl.BlockSpec((1, D), lambda i: (0, 0)),
            pl.BlockSpec((1, D), lambda i: (0, 0)),
            pl.BlockSpec((D, 2 * D), lambda i: (0, 0)),
            pl.BlockSpec((1, 2 * D), lambda i: (0, 0)),
            pl.BlockSpec((2 * D, D), lambda i: (0, 0)),
            pl.BlockSpec((1, D), lambda i: (0, 0)),
        ],
        out_specs=[
            pl.BlockSpec((_BLK, D), lambda i: (i, 0)),
            pl.BlockSpec((8, D), lambda i: (0, 0)),
        ],
        out_shape=[
            jax.ShapeDtypeStruct((n, D), jnp.float32),
            jax.ShapeDtypeStruct((8, D), jnp.float32),
        ],
    )(x1, st, g1b, be1b, w1t, b1b, w2t, b2b)


def _bn2_body(x2_ref, st2_ref, g2_ref, be2_ref, out_ref):
    inv_n = 1.0 / N_NODES
    mu = st2_ref[0:1, :] * inv_n
    var = st2_ref[1:2, :] * inv_n - mu * mu
    inv = lax.rsqrt(var + 1e-5)
    out_ref[...] = (x2_ref[...] - mu) * (inv * g2_ref[...]) + be2_ref[...]


def _bn2(x2, st2, g2b, be2b):
    n = x2.shape[0]
    grid = n // _BLK
    return pl.pallas_call(
        _bn2_body,
        grid=(grid,),
        in_specs=[
            pl.BlockSpec((_BLK, D), lambda i: (i, 0)),
            pl.BlockSpec((8, D), lambda i: (0, 0)),
            pl.BlockSpec((1, D), lambda i: (0, 0)),
            pl.BlockSpec((1, D), lambda i: (0, 0)),
        ],
        out_specs=pl.BlockSpec((_BLK, D), lambda i: (i, 0)),
        out_shape=jax.ShapeDtypeStruct((n, D), jnp.float32),
    )(x2, st2, g2b, be2b)


def kernel(h, edge_index, Wq, Wk, Wv, Wo, bo, W1, b1, W2, b2, g1, be1, g2, be2):
    cp = jnp.arange(D, dtype=jnp.int32)
    perm_orig = (cp % H) * DH + cp // H
    wqt = Wq.T[:, perm_orig]
    wkt = Wk.T[:, perm_orig]
    wvt = Wv.T[:, perm_orig]
    wot = Wo.T[perm_orig, :]
    w1t = W1.T
    w2t = W2.T
    bo2 = bo.reshape(1, D)
    b1b = b1.reshape(1, 2 * D)
    b2b = b2.reshape(1, D)
    g1b = g1.reshape(1, D)
    be1b = be1.reshape(1, D)
    g2b = g2.reshape(1, D)
    be2b = be2.reshape(1, D)
    rmat = (cp[None, :] % H
            == jnp.arange(H, dtype=jnp.int32)[:, None]).astype(jnp.float32)

    q, k, v = _qkv(h, wqt, wkt, wvt)
    acc_wv, acc_z = _edge_aggregate(q, k, v, edge_index[0], edge_index[1])
    zf = acc_z.reshape(NC, N_PAD, H)
    x1, st = _attn_out(acc_wv, zf, h, rmat, wot, bo2)
    x2, st2 = _ffn(x1, st, g1b, be1b, w1t, b1b, w2t, b2b)
    return _bn2(x2, st2, g2b, be2b)


# speedup vs baseline: 2.2169x; 1.0006x over previous
import jax
import jax.numpy as jnp
from jax import lax
from jax.experimental import pallas as pl
from jax.experimental.pallas import tpu as pltpu
from jax.experimental.pallas import tpu_sc as plsc

N_NODES = 10000
N_PAD = 10240
D = 128
H = 8
DH = 16

NC = 2
NS = 16
NW = NC * NS

ZROWS = N_PAD // 16

CHUNK = 40
SUP = 10
IO_ROWS = 40


def _sc_edge_kernel(q_hbm, k_hbm, v_hbm, src_hbm, dst_hbm,
                    out_wv, out_z,
                    srcb, dstb, kqv0, kqv1, vbuf, contrib, contrib_z,
                    svbuf, dst_sc0, dst_sc1, zrow0, zrow1, acc_wv, acc_z,
                    sem_g0, sem_g1, sem_v, sem_s, sem_z, sem_zf):
    c = lax.axis_index("c")
    s = lax.axis_index("s")
    w = s * NC + c

    e_total = src_hbm.shape[0]
    e_per_tile = e_total // NW
    n_super = e_per_tile // (SUP * CHUNK)
    rows_per_tile = N_PAD // NS
    zrows_per_tile = ZROWS // NS

    lane = lax.iota(jnp.int32, 16)
    zero16 = jnp.zeros((16,), jnp.float32)
    kq = (kqv0, kqv1)
    sem_g = (sem_g0, sem_g1)
    dst_scb = (dst_sc0, dst_sc1)
    zrowb = (zrow0, zrow1)
    iobuf = kqv0

    @pl.loop(0, IO_ROWS * 8)
    def _z(i):
        iobuf[i // 8, pl.ds((i % 8) * 16, 16)] = zero16

    io0 = iobuf.at[pl.ds(0, IO_ROWS)]

    @pl.loop(0, rows_per_tile // IO_ROWS)
    def _z2(j):
        pltpu.sync_copy(io0, acc_wv.at[pl.ds(s * rows_per_tile + j * IO_ROWS, IO_ROWS)])

    pltpu.sync_copy(iobuf.at[pl.ds(0, zrows_per_tile)],
                    acc_z.at[pl.ds(s * zrows_per_tile, zrows_per_tile)])

    plsc.subcore_barrier()

    row_off = lane // H
    head = lane % H
    cols = tuple(head + ((d + row_off) % DH) * H for d in range(DH))
    ebase0 = w * e_per_tile

    def issue_gathers(sub, sbase):
        b = sub % 2
        buf = kq[b]
        sem = sem_g[b]
        so = sub * CHUNK
        ck = pltpu.async_copy(k_hbm.at[srcb.at[pl.ds(so, CHUNK)]],
                              buf.at[pl.ds(0, CHUNK)], sem)
        cq = pltpu.async_copy(q_hbm.at[dstb.at[pl.ds(so, CHUNK)]],
                              buf.at[pl.ds(CHUNK, CHUNK)], sem)
        return (ck, cq)

    def compute_chunk(sub, pend_sc):
        b = sub % 2
        buf = kq[b]
        kbuf = buf.at[pl.ds(0, CHUNK)]
        qbuf = buf.at[pl.ds(CHUNK, CHUNK)]
        dst_sc = dst_scb[b]
        zrow_v = zrowb[b]
        so = sub * CHUNK
        cv = pltpu.async_copy(v_hbm.at[srcb.at[pl.ds(so, CHUNK)]],
                              vbuf, sem_v)
        for t in range(CHUNK // 16):
            dv = dstb[pl.ds(so + t * 16, 16)]
            dst_sc[pl.ds(t * 16, 16)] = dv
            zrow_v[pl.ds(t * 16, 16)] = dv >> 4
        rem = CHUNK % 16
        if rem:
            ridx = jnp.minimum(lane + (CHUNK // 16) * 16, CHUNK - 1)
            dvr = plsc.load_gather(dstb, [so + ridx])
            plsc.store_scatter(dst_sc, [ridx], dvr, mask=lane < rem)
            plsc.store_scatter(zrow_v, [ridx], dvr >> 4, mask=lane < rem)

        @pl.loop(0, CHUNK // 4)
        def _grp(i):
            rows0 = row_off + 4 * i
            rows1 = row_off + 4 * i + 2
            sa0 = jnp.zeros((16,), jnp.float32)
            sb0 = jnp.zeros((16,), jnp.float32)
            sa1 = jnp.zeros((16,), jnp.float32)
            sb1 = jnp.zeros((16,), jnp.float32)
            for d in range(0, DH, 2):
                sa0 = sa0 + (plsc.load_gather(kbuf, [rows0, cols[d]])
                             * plsc.load_gather(qbuf, [rows0, cols[d]]))
                sa1 = sa1 + (plsc.load_gather(kbuf, [rows1, cols[d]])
                             * plsc.load_gather(qbuf, [rows1, cols[d]]))
                sb0 = sb0 + (plsc.load_gather(kbuf, [rows0, cols[d + 1]])
                             * plsc.load_gather(qbuf, [rows0, cols[d + 1]]))
                sb1 = sb1 + (plsc.load_gather(kbuf, [rows1, cols[d + 1]])
                             * plsc.load_gather(qbuf, [rows1, cols[d + 1]]))
            sval0 = jnp.exp(jnp.clip((sa0 + sb0) * 0.25, -5.0, 5.0))
            sval1 = jnp.exp(jnp.clip((sa1 + sb1) * 0.25, -5.0, 5.0))
            svbuf[pl.ds(i * 32, 16)] = sval0
            svbuf[pl.ds(i * 32 + 16, 16)] = sval1

        cv.wait()
        for d_ in pend_sc:
            d_.wait()
        zf = pltpu.async_copy(acc_wv.at[pl.ds(N_PAD - CHUNK, CHUNK)],
                              contrib_z, sem_zf)

        @pl.loop(0, CHUNK // 4)
        def _vph(i):
            rows0 = row_off + 4 * i
            rows1 = row_off + 4 * i + 2
            sval0 = svbuf[pl.ds(i * 32, 16)]
            sval1 = svbuf[pl.ds(i * 32 + 16, 16)]
            for d in range(DH):
                v0 = plsc.load_gather(vbuf, [rows0, cols[d]])
                v1 = plsc.load_gather(vbuf, [rows1, cols[d]])
                plsc.store_scatter(contrib, [rows0, cols[d]], v0 * sval0)
                plsc.store_scatter(contrib, [rows1, cols[d]], v1 * sval1)

        zf.wait()

        @pl.loop(0, CHUNK)
        def _zed(e):
            dsp = plsc.load_gather(dst_sc, [jnp.full((16,), e, jnp.int32)])
            sv = svbuf[pl.ds(e * 8, 16)]
            plsc.store_scatter(
                contrib_z,
                [jnp.full((16,), e, jnp.int32), (dsp & 15) * 8 + lane],
                sv, mask=lane < 8)

        cw = pltpu.async_copy(contrib, acc_wv.at[dst_sc], sem_s, add=True)
        cz = pltpu.async_copy(contrib_z, acc_z.at[zrow_v], sem_z, add=True)
        return (cw, cz)

    @pl.loop(0, n_super)
    def _sup(sp):
        sbase = ebase0 + sp * (SUP * CHUNK)
        pltpu.sync_copy(src_hbm.at[pl.ds(sbase, SUP * CHUNK)],
                        srcb.at[pl.ds(0, SUP * CHUNK)])
        pltpu.sync_copy(dst_hbm.at[pl.ds(sbase, SUP * CHUNK)],
                        dstb.at[pl.ds(0, SUP * CHUNK)])
        pend = issue_gathers(0, sbase)
        psc = ()
        for sub in range(SUP):
            nxt = issue_gathers(sub + 1, sbase) if sub + 1 < SUP else None
            for d_ in pend:
                d_.wait()
            pend = nxt
            psc = compute_chunk(sub, psc)
        for d_ in psc:
            d_.wait()

    plsc.subcore_barrier()

    @pl.loop(0, rows_per_tile // IO_ROWS)
    def _out(j):
        r0 = s * rows_per_tile + j * IO_ROWS
        pltpu.sync_copy(acc_wv.at[pl.ds(r0, IO_ROWS)], io0)
        pltpu.sync_copy(io0, out_wv.at[c, pl.ds(r0, IO_ROWS)])

    zr0 = s * zrows_per_tile
    pltpu.sync_copy(acc_z.at[pl.ds(zr0, zrows_per_tile)],
                    iobuf.at[pl.ds(0, zrows_per_tile)])
    pltpu.sync_copy(iobuf.at[pl.ds(0, zrows_per_tile)],
                    out_z.at[c, pl.ds(zr0, zrows_per_tile)])


def _edge_aggregate(q, k, v, src, dst):
    fn = pl.kernel(
        _sc_edge_kernel,
        out_type=[
            jax.ShapeDtypeStruct((NC, N_PAD, D), jnp.float32),
            jax.ShapeDtypeStruct((NC, ZROWS, D), jnp.float32),
        ],
        mesh=plsc.VectorSubcoreMesh(core_axis_name="c", subcore_axis_name="s",
                                    num_cores=NC, num_subcores=NS),
        compiler_params=pltpu.CompilerParams(needs_layout_passes=False),
        scratch_types=[
            pltpu.VMEM((SUP * CHUNK + 16,), jnp.int32),
            pltpu.VMEM((SUP * CHUNK + 16,), jnp.int32),
            pltpu.VMEM((2 * CHUNK, D), jnp.float32),
            pltpu.VMEM((2 * CHUNK, D), jnp.float32),
            pltpu.VMEM((CHUNK, D), jnp.float32),
            pltpu.VMEM((CHUNK, D), jnp.float32),
            pltpu.VMEM((CHUNK, D), jnp.float32),
            pltpu.VMEM((CHUNK * 8 + 8,), jnp.float32),
            pltpu.VMEM((CHUNK,), jnp.int32),
            pltpu.VMEM((CHUNK,), jnp.int32),
            pltpu.VMEM((CHUNK,), jnp.int32),
            pltpu.VMEM((CHUNK,), jnp.int32),
            pltpu.VMEM_SHARED((N_PAD, D), jnp.float32),
            pltpu.VMEM_SHARED((ZROWS, D), jnp.float32),
            pltpu.SemaphoreType.DMA,
            pltpu.SemaphoreType.DMA,
            pltpu.SemaphoreType.DMA,
            pltpu.SemaphoreType.DMA,
            pltpu.SemaphoreType.DMA,
            pltpu.SemaphoreType.DMA,
        ],
    )
    return fn(q, k, v, src, dst)


_BLK = 1000


def _qkv_body(h_ref, wqt_ref, wkt_ref, wvt_ref, q_ref, k_ref, v_ref):
    hb = h_ref[...]
    q_ref[...] = jnp.dot(hb, wqt_ref[...], preferred_element_type=jnp.float32)
    k_ref[...] = jnp.dot(hb, wkt_ref[...], preferred_element_type=jnp.float32)
    v_ref[...] = jnp.dot(hb, wvt_ref[...], preferred_element_type=jnp.float32)


def _qkv(h, wqt, wkt, wvt):
    n = h.shape[0]
    grid = n // _BLK
    return pl.pallas_call(
        _qkv_body,
        grid=(grid,),
        in_specs=[
            pl.BlockSpec((_BLK, D), lambda i: (i, 0)),
            pl.BlockSpec((D, D), lambda i: (0, 0)),
            pl.BlockSpec((D, D), lambda i: (0, 0)),
            pl.BlockSpec((D, D), lambda i: (0, 0)),
        ],
        out_specs=[
            pl.BlockSpec((_BLK, D), lambda i: (i, 0)),
            pl.BlockSpec((_BLK, D), lambda i: (i, 0)),
            pl.BlockSpec((_BLK, D), lambda i: (i, 0)),
        ],
        out_shape=[jax.ShapeDtypeStruct((n, D), jnp.float32)] * 3,
    )(h, wqt, wkt, wvt)


def _attn_out_body(wv_ref, z_ref, h_ref, r_ref, wot_ref, bo_ref, x1_ref,
                   st_ref):
    wv = wv_ref[0] + wv_ref[1]
    z = z_ref[0] + z_ref[1]
    zr = jnp.dot(z, r_ref[...], preferred_element_type=jnp.float32)
    head_out = wv / zr
    x = (jnp.dot(head_out, wot_ref[...], preferred_element_type=jnp.float32)
         + bo_ref[...] + h_ref[...])
    x1_ref[...] = x

    @pl.when(pl.program_id(0) == 0)
    def _():
        st_ref[...] = jnp.zeros_like(st_ref)

    st_ref[0:1, :] += jnp.sum(x, axis=0, keepdims=True)
    st_ref[1:2, :] += jnp.sum(x * x, axis=0, keepdims=True)


def _attn_out(acc_wv, zf, h, rmat, wot, bo2):
    n = h.shape[0]
    grid = n // _BLK
    return pl.pallas_call(
        _attn_out_body,
        grid=(grid,),
        in_specs=[
            pl.BlockSpec((NC, _BLK, D), lambda i: (0, i, 0)),
            pl.BlockSpec((NC, _BLK, H), lambda i: (0, i, 0)),
            pl.BlockSpec((_BLK, D), lambda i: (i, 0)),
            pl.BlockSpec((H, D), lambda i: (0, 0)),
            pl.BlockSpec((D, D), lambda i: (0, 0)),
            pl.BlockSpec((1, D), lambda i: (0, 0)),
        ],
        out_specs=[
            pl.BlockSpec((_BLK, D), lambda i: (i, 0)),
            pl.BlockSpec((8, D), lambda i: (0, 0)),
        ],
        out_shape=[
            jax.ShapeDtypeStruct((n, D), jnp.float32),
            jax.ShapeDtypeStruct((8, D), jnp.float32),
        ],
    )(acc_wv, zf, h, rmat, wot, bo2)


def _ffn_body(x1_ref, st_ref, g1_ref, be1_ref, w1t_ref, b1_ref, w2t_ref,
              b2_ref, x2_ref, st2_ref):
    inv_n = 1.0 / N_NODES
    mu = st_ref[0:1, :] * inv_n
    var = st_ref[1:2, :] * inv_n - mu * mu
    inv = lax.rsqrt(var + 1e-5)
    xn = (x1_ref[...] - mu) * (inv * g1_ref[...]) + be1_ref[...]
    t = jnp.maximum(
        jnp.dot(xn, w1t_ref[...], preferred_element_type=jnp.float32)
        + b1_ref[...], 0.0)
    y = jnp.dot(t, w2t_ref[...], preferred_element_type=jnp.float32) + b2_ref[...]
    x2 = xn + y
    x2_ref[...] = x2

    @pl.when(pl.program_id(0) == 0)
    def _():
        st2_ref[...] = jnp.zeros_like(st2_ref)

    st2_ref[0:1, :] += jnp.sum(x2, axis=0, keepdims=True)
    st2_ref[1:2, :] += jnp.sum(x2 * x2, axis=0, keepdims=True)


def _ffn(x1, st, g1b, be1b, w1t, b1b, w2t, b2b):
    n = x1.shape[0]
    grid = n // _BLK
    return pl.pallas_call(
        _ffn_body,
        grid=(grid,),
        in_specs=[
            pl.BlockSpec((_BLK, D), lambda i: (i, 0)),
            pl.BlockSpec((8, D), lambda i: (0, 0)),
            pl.BlockSpec((1, D), lambda i: (0, 0)),
            pl.BlockSpec((1, D), lambda i: (0, 0)),
            pl.BlockSpec((D, 2 * D), lambda i: (0, 0)),
            pl.BlockSpec((1, 2 * D), lambda i: (0, 0)),
            pl.BlockSpec((2 * D, D), lambda i: (0, 0)),
            pl.BlockSpec((1, D), lambda i: (0, 0)),
        ],
        out_specs=[
            pl.BlockSpec((_BLK, D), lambda i: (i, 0)),
            pl.BlockSpec((8, D), lambda i: (0, 0)),
        ],
        out_shape=[
            jax.ShapeDtypeStruct((n, D), jnp.float32),
            jax.ShapeDtypeStruct((8, D), jnp.float32),
        ],
    )(x1, st, g1b, be1b, w1t, b1b, w2t, b2b)


def _bn2_body(x2_ref, st2_ref, g2_ref, be2_ref, out_ref):
    inv_n = 1.0 / N_NODES
    mu = st2_ref[0:1, :] * inv_n
    var = st2_ref[1:2, :] * inv_n - mu * mu
    inv = lax.rsqrt(var + 1e-5)
    out_ref[...] = (x2_ref[...] - mu) * (inv * g2_ref[...]) + be2_ref[...]


def _bn2(x2, st2, g2b, be2b):
    n = x2.shape[0]
    grid = n // _BLK
    return pl.pallas_call(
        _bn2_body,
        grid=(grid,),
        in_specs=[
            pl.BlockSpec((_BLK, D), lambda i: (i, 0)),
            pl.BlockSpec((8, D), lambda i: (0, 0)),
            pl.BlockSpec((1, D), lambda i: (0, 0)),
            pl.BlockSpec((1, D), lambda i: (0, 0)),
        ],
        out_specs=pl.BlockSpec((_BLK, D), lambda i: (i, 0)),
        out_shape=jax.ShapeDtypeStruct((n, D), jnp.float32),
    )(x2, st2, g2b, be2b)


def kernel(h, edge_index, Wq, Wk, Wv, Wo, bo, W1, b1, W2, b2, g1, be1, g2, be2):
    cp = jnp.arange(D, dtype=jnp.int32)
    perm_orig = (cp % H) * DH + cp // H
    wqt = Wq.T[:, perm_orig]
    wkt = Wk.T[:, perm_orig]
    wvt = Wv.T[:, perm_orig]
    wot = Wo.T[perm_orig, :]
    w1t = W1.T
    w2t = W2.T
    bo2 = bo.reshape(1, D)
    b1b = b1.reshape(1, 2 * D)
    b2b = b2.reshape(1, D)
    g1b = g1.reshape(1, D)
    be1b = be1.reshape(1, D)
    g2b = g2.reshape(1, D)
    be2b = be2.reshape(1, D)
    rmat = (cp[None, :] % H
            == jnp.arange(H, dtype=jnp.int32)[:, None]).astype(jnp.float32)

    q, k, v = _qkv(h, wqt, wkt, wvt)
    acc_wv, acc_z = _edge_aggregate(q, k, v, edge_index[0], edge_index[1])
    zf = acc_z.reshape(NC, N_PAD, H)
    x1, st = _attn_out(acc_wv, zf, h, rmat, wot, bo2)
    x2, st2 = _ffn(x1, st, g1b, be1b, w1t, b1b, w2t, b2b)
    return _bn2(x2, st2, g2b, be2b)
